# Initial kernel scaffold; baseline (speedup 1.0000x reference)
import jax
import jax.numpy as jnp
from jax.experimental import pallas as pl


def kernel(input, target, table):
    raise NotImplementedError("write your pallas kernel here")


# speedup vs baseline: 1.9226x; 1.9226x over previous
import functools

import jax
import jax.numpy as jnp
from jax import lax
from jax.experimental import pallas as pl
from jax.experimental.pallas import tpu as pltpu
from jax.experimental.pallas import tpu_sc as plsc

V = 65
VP = 72
N = 64 * 2048
NW = 32
PER_W = N // NW
CHUNK = 512
NCHUNK = PER_W // CHUNK
L = 16


def _nll_body(table_ref, nll_ref):
    t = table_ref[...]
    m = jnp.max(t, axis=1, keepdims=True)
    s = jnp.sum(jnp.exp(t - m), axis=1, keepdims=True)
    nll_ref[...] = (m + jnp.log(s)) - t


def _nll_table(table):
    return pl.pallas_call(
        _nll_body,
        out_shape=jax.ShapeDtypeStruct((V, V), jnp.float32),
    )(table).reshape(V * V)


def _sc_body(tablep, nllf_hbm, inp_hbm, tgt_hbm, out_hbm, part_hbm,
             inp_v, tgt_v, nll_v, rows_v, acc_v, sem):
    wid = lax.axis_index("s") * 2 + lax.axis_index("c")
    base = wid * PER_W
    pltpu.sync_copy(inp_hbm.at[pl.ds(base, PER_W)], inp_v)
    pltpu.sync_copy(tgt_hbm.at[pl.ds(base, PER_W)], tgt_v)
    pltpu.sync_copy(nllf_hbm, nll_v)

    def loss_body(i, acc):
        iv = inp_v[pl.ds(i * L, L)]
        tv = tgt_v[pl.ds(i * L, L)]
        return acc + plsc.load_gather(nll_v, [iv * V + tv])

    acc = lax.fori_loop(0, PER_W // L, loss_body, jnp.zeros((L,), jnp.float32))
    acc_v[...] = acc
    pltpu.sync_copy(acc_v, part_hbm.at[wid])

    def gather_body(c, carry):
        idx = inp_v.at[pl.ds(c * CHUNK, CHUNK)]
        pltpu.async_copy(tablep.at[idx], rows_v, sem).wait()
        pltpu.sync_copy(rows_v, out_hbm.at[pl.ds(base + c * CHUNK, CHUNK)])
        return carry

    lax.fori_loop(0, NCHUNK, gather_body, 0)


_sc_call = functools.partial(
    pl.kernel,
    out_type=(jax.ShapeDtypeStruct((N, VP), jnp.float32),
              jax.ShapeDtypeStruct((NW, L), jnp.float32)),
    mesh=plsc.VectorSubcoreMesh(core_axis_name="c", subcore_axis_name="s"),
    scratch_types=[
        pltpu.VMEM((PER_W,), jnp.int32),
        pltpu.VMEM((PER_W,), jnp.int32),
        pltpu.VMEM((V * V,), jnp.float32),
        pltpu.VMEM((CHUNK, VP), jnp.float32),
        pltpu.VMEM((L,), jnp.float32),
        pltpu.SemaphoreType.DMA,
    ],
    compiler_params=pltpu.CompilerParams(
        needs_layout_passes=False, use_tc_tiling_on_sc=False),
)(_sc_body)


def _loss_reduce_body(p_ref, o_ref):
    o_ref[...] = (jnp.sum(p_ref[...]) / N).reshape(1, 1)


def _loss_reduce(partials):
    return pl.pallas_call(
        _loss_reduce_body,
        out_shape=jax.ShapeDtypeStruct((1, 1), jnp.float32),
    )(partials)


def kernel(input, target, table):
    nllf = _nll_table(table)
    tablep = jnp.pad(table, ((0, 0), (0, VP - V)))
    out_pad, partials = _sc_call(
        tablep, nllf, input.reshape(N), target.reshape(N))
    logits = out_pad[:, :V].reshape(input.shape + (V,))
    loss = _loss_reduce(partials)[0, 0]
    return (logits, loss)


# speedup vs baseline: 3.3171x; 1.7253x over previous
import functools

import jax
import jax.numpy as jnp
from jax import lax
from jax.experimental import pallas as pl
from jax.experimental.pallas import tpu as pltpu
from jax.experimental.pallas import tpu_sc as plsc

V = 65
VP = 72
N = 64 * 2048
NW = 32
PER_W = N // NW
CHUNK = 512
NCHUNK = PER_W // CHUNK
L = 16


def _nll_body(table_ref, nll_ref):
    t = table_ref[...]
    m = jnp.max(t, axis=1, keepdims=True)
    s = jnp.sum(jnp.exp(t - m), axis=1, keepdims=True)
    nll_ref[...] = (m + jnp.log(s)) - t


def _nll_table(table):
    return pl.pallas_call(
        _nll_body,
        out_shape=jax.ShapeDtypeStruct((V, V), jnp.float32),
    )(table).reshape(V * V)


def _sc_body(tablep, nllf_hbm, inp_hbm, tgt_hbm, out_hbm, part_hbm,
             inp_v, tgt_v, nll_v, tab_v, rows_a, rows_b, acc_v,
             gsem_a, gsem_b, wsem_a, wsem_b):
    wid = lax.axis_index("s") * 2 + lax.axis_index("c")
    base = wid * PER_W
    pltpu.sync_copy(inp_hbm.at[pl.ds(base, PER_W)], inp_v)
    pltpu.sync_copy(tgt_hbm.at[pl.ds(base, PER_W)], tgt_v)
    pltpu.sync_copy(nllf_hbm, nll_v)

    @pl.when(lax.axis_index("s") == 0)
    def _():
        pltpu.sync_copy(tablep, tab_v)

    plsc.subcore_barrier()

    def loss_body(i, acc):
        iv = inp_v[pl.ds(i * L, L)]
        tv = tgt_v[pl.ds(i * L, L)]
        return acc + plsc.load_gather(nll_v, [iv * V + tv])

    acc = lax.fori_loop(0, PER_W // L, loss_body, jnp.zeros((L,), jnp.float32))
    acc_v[...] = acc
    pltpu.sync_copy(acc_v, part_hbm.at[wid])

    bufs = (rows_a, rows_b)
    gsems = (gsem_a, gsem_b)
    wsems = (wsem_a, wsem_b)

    def gather_start(c):
        idx = inp_v.at[pl.ds(c * CHUNK, CHUNK)]
        return pltpu.async_copy(tab_v.at[idx], bufs[c % 2], gsems[c % 2])

    def write_start(c):
        return pltpu.async_copy(
            bufs[c % 2], out_hbm.at[pl.ds(base + c * CHUNK, CHUNK)],
            wsems[c % 2])

    gcp = [None, None]
    wcp = [None, None]
    gcp[0] = gather_start(0)
    for c in range(NCHUNK):
        if c + 1 < NCHUNK:
            if wcp[(c + 1) % 2] is not None:
                wcp[(c + 1) % 2].wait()
            gcp[(c + 1) % 2] = gather_start(c + 1)
        gcp[c % 2].wait()
        wcp[c % 2] = write_start(c)
    wcp[(NCHUNK - 2) % 2].wait()
    wcp[(NCHUNK - 1) % 2].wait()


_sc_call = functools.partial(
    pl.kernel,
    out_type=(jax.ShapeDtypeStruct((N, VP), jnp.float32),
              jax.ShapeDtypeStruct((NW, L), jnp.float32)),
    mesh=plsc.VectorSubcoreMesh(core_axis_name="c", subcore_axis_name="s"),
    scratch_types=[
        pltpu.VMEM((PER_W,), jnp.int32),
        pltpu.VMEM((PER_W,), jnp.int32),
        pltpu.VMEM((V * V,), jnp.float32),
        pltpu.VMEM_SHARED((V, VP), jnp.float32),
        pltpu.VMEM((CHUNK, VP), jnp.float32),
        pltpu.VMEM((CHUNK, VP), jnp.float32),
        pltpu.VMEM((L,), jnp.float32),
        pltpu.SemaphoreType.DMA,
        pltpu.SemaphoreType.DMA,
        pltpu.SemaphoreType.DMA,
        pltpu.SemaphoreType.DMA,
    ],
    compiler_params=pltpu.CompilerParams(
        needs_layout_passes=False, use_tc_tiling_on_sc=False),
)(_sc_body)


def _loss_reduce_body(p_ref, o_ref):
    o_ref[...] = (jnp.sum(p_ref[...]) / N).reshape(1, 1)


def _loss_reduce(partials):
    return pl.pallas_call(
        _loss_reduce_body,
        out_shape=jax.ShapeDtypeStruct((1, 1), jnp.float32),
    )(partials)


def kernel(input, target, table):
    nllf = _nll_table(table)
    tablep = jnp.pad(table, ((0, 0), (0, VP - V)))
    out_pad, partials = _sc_call(
        tablep, nllf, input.reshape(N), target.reshape(N))
    logits = out_pad[:, :V].reshape(input.shape + (V,))
    loss = _loss_reduce(partials)[0, 0]
    return (logits, loss)


# speedup vs baseline: 5.4398x; 1.6399x over previous
import functools

import jax
import jax.numpy as jnp
from jax import lax
from jax.experimental import pallas as pl
from jax.experimental.pallas import tpu as pltpu
from jax.experimental.pallas import tpu_sc as plsc

V = 65
VS = 72
N = 64 * 2048
NW = 32
PER_W = N // NW
CHUNK = 256
NCHUNK = PER_W // CHUNK
L = 16


def _nll_body(table_ref, nll_ref):
    t = table_ref[...]
    m = jnp.max(t, axis=1, keepdims=True)
    s = jnp.sum(jnp.exp(t - m), axis=1, keepdims=True)
    nll_ref[...] = (m + jnp.log(s)) - t


def _nll_table(table):
    return pl.pallas_call(
        _nll_body,
        out_shape=jax.ShapeDtypeStruct((V, V), jnp.float32),
    )(table).reshape(V * V)


def _sc_body(tabf_hbm, nllf_hbm, inp_hbm, tgt_hbm, out_hbm, part_hbm,
             inp_v, tgt_v, nll_v, tmp_v, tab_s, rows_a, rows_b, acc_v,
             fsem, gsem_a, gsem_b, wsem_a, wsem_b):
    wid = lax.axis_index("s") * 2 + lax.axis_index("c")
    base = wid * PER_W
    pltpu.sync_copy(inp_hbm.at[pl.ds(base, PER_W)], inp_v)
    pltpu.sync_copy(tgt_hbm.at[pl.ds(base, PER_W)], tgt_v)
    pltpu.sync_copy(nllf_hbm, nll_v)

    fills = [
        pltpu.async_copy(tabf_hbm.at[pl.ds(VS * r, V)], tmp_v.at[r], fsem)
        for r in range(V)
    ]

    def loss_body(i, acc):
        iv = inp_v[pl.ds(i * L, L)]
        tv = tgt_v[pl.ds(i * L, L)]
        return acc + plsc.load_gather(nll_v, [iv * V + tv])

    acc = lax.fori_loop(0, PER_W // L, loss_body, jnp.zeros((L,), jnp.float32))
    acc_v[...] = acc
    pltpu.sync_copy(acc_v, part_hbm.at[pl.ds(wid * L, L)])
    for cp in fills:
        cp.wait()
    pltpu.sync_copy(tmp_v, tab_s)

    bufs = (rows_a, rows_b)
    gsems = (gsem_a, gsem_b)
    wsems = (wsem_a, wsem_b)

    def gather_start(c):
        idx = inp_v.at[pl.ds(c * CHUNK, CHUNK)]
        return pltpu.async_copy(tab_s.at[idx], bufs[c % 2], gsems[c % 2])

    def write_start(c):
        return pltpu.async_copy(
            bufs[c % 2], out_hbm.at[pl.ds(base + c * CHUNK, CHUNK)],
            wsems[c % 2])

    gcp = [None, None]
    wcp = [None, None]
    gcp[0] = gather_start(0)
    for c in range(NCHUNK):
        if c + 1 < NCHUNK:
            if wcp[(c + 1) % 2] is not None:
                wcp[(c + 1) % 2].wait()
            gcp[(c + 1) % 2] = gather_start(c + 1)
        gcp[c % 2].wait()
        wcp[c % 2] = write_start(c)
    wcp[(NCHUNK - 2) % 2].wait()
    wcp[(NCHUNK - 1) % 2].wait()


_sc_call = functools.partial(
    pl.kernel,
    out_type=(jax.ShapeDtypeStruct((N, V), jnp.float32),
              jax.ShapeDtypeStruct((NW * L,), jnp.float32)),
    mesh=plsc.VectorSubcoreMesh(core_axis_name="c", subcore_axis_name="s"),
    scratch_types=[
        pltpu.VMEM((PER_W,), jnp.int32),
        pltpu.VMEM((PER_W,), jnp.int32),
        pltpu.VMEM((V * V,), jnp.float32),
        pltpu.VMEM((V, V), jnp.float32),
        pltpu.VMEM_SHARED((V, V), jnp.float32),
        pltpu.VMEM((CHUNK, V), jnp.float32),
        pltpu.VMEM((CHUNK, V), jnp.float32),
        pltpu.VMEM((L,), jnp.float32),
        pltpu.SemaphoreType.DMA,
        pltpu.SemaphoreType.DMA,
        pltpu.SemaphoreType.DMA,
        pltpu.SemaphoreType.DMA,
        pltpu.SemaphoreType.DMA,
    ],
    compiler_params=pltpu.CompilerParams(needs_layout_passes=False),
)(_sc_body)


def _loss_reduce_body(p_ref, o_ref):
    o_ref[...] = (jnp.sum(p_ref[...]) / N).reshape(1, 1)


def _loss_reduce(partials):
    return pl.pallas_call(
        _loss_reduce_body,
        out_shape=jax.ShapeDtypeStruct((1, 1), jnp.float32),
    )(partials)


def kernel(input, target, table):
    nllf = _nll_table(table)
    tabf = jnp.pad(table, ((0, 0), (0, VS - V))).reshape(V * VS)
    logits_flat, partials = _sc_call(
        tabf, nllf, input.reshape(N), target.reshape(N))
    logits = logits_flat.reshape(input.shape + (V,))
    loss = _loss_reduce(partials.reshape(NW, L))[0, 0]
    return (logits, loss)
